# Initial kernel scaffold; baseline (speedup 1.0000x reference)
import jax
import jax.numpy as jnp
from jax.experimental import pallas as pl


def kernel(points, features, W1, gamma1, beta1, Wsc, gamma_sc, beta_sc):
    raise NotImplementedError("write your pallas kernel here")


# speedup vs baseline: 3.8082x; 3.8082x over previous
import functools

import jax
import jax.numpy as jnp
from jax import lax
from jax.experimental import pallas as pl
from jax.experimental.pallas import tpu as pltpu
from jax.experimental.pallas import tpu_sc as plsc

_B, _N, _K, _DIN, _DOUT = 32, 1024, 16, 64, 128
_EPS = 1e-5
_NEG_INF = float("-inf")


def _event_body(pts_ref, x_ref, at_ref, bmt_ref, wsct_ref,
                idx_ref, vt_ref, ut_ref, ht_ref, st_ref,
                arr_ref, madj_ref):
    b = pl.program_id(0)
    pts = pts_ref[0]
    x = x_ref[0]

    ptst = pts.T
    inner = jax.lax.dot_general(
        ptst, pts, (((1,), (0,)), ((), ())),
        preferred_element_type=jnp.float32)
    r_col = jnp.sum(ptst * ptst, axis=1, keepdims=True)
    r_row = jnp.sum(pts * pts, axis=0)[None, :]
    arr_ref[...] = 2.0 * inner - r_col - r_row
    madj_ref[...] = jnp.zeros((_N, _N), dtype=jnp.float32)
    idx_ref[0] = jnp.zeros((_N, _K), dtype=jnp.int32)

    ci = lax.broadcasted_iota(jnp.int32, (_N, _N), 1)
    ki = lax.broadcasted_iota(jnp.int32, (_N, _K), 1)

    def step(t, carry):
        arr = arr_ref[...]
        mx = jnp.max(arr, axis=1, keepdims=True)
        am = jnp.min(jnp.where(arr == mx, ci, _N), axis=1,
                     keepdims=True)
        hit = ci == am
        arr_ref[...] = jnp.where(hit, _NEG_INF, arr)

        @pl.when(t > 0)
        def _():
            madj_ref[...] += hit.astype(jnp.float32)
            idx_ref[0] = jnp.where(ki == (t - 1), am, idx_ref[0])

        return carry

    lax.fori_loop(0, _K + 1, step, 0)
    idxm = idx_ref[0]
    m_adj = madj_ref[...]

    xt = x.T
    ut = jnp.dot(xt, at_ref[...], preferred_element_type=jnp.float32)
    vt = jnp.dot(xt, bmt_ref[...], preferred_element_type=jnp.float32)
    ht = jnp.dot(xt, wsct_ref[...], preferred_element_type=jnp.float32)

    s_nb = jnp.dot(m_adj, vt, preferred_element_type=jnp.float32)
    q_nb = jnp.dot(m_adj, vt * vt, preferred_element_type=jnp.float32)

    sum_g = _K * jnp.sum(ut, axis=0) + jnp.sum(s_nb, axis=0)
    ssq_g = (_K * jnp.sum(ut * ut, axis=0)
             + 2.0 * jnp.sum(ut * s_nb, axis=0)
             + jnp.sum(q_nb, axis=0))
    sum_h = jnp.sum(ht, axis=0)
    ssq_h = jnp.sum(ht * ht, axis=0)
    st = jnp.concatenate([
        sum_g[None, :], ssq_g[None, :], sum_h[None, :], ssq_h[None, :],
        jnp.zeros((4, _DOUT), dtype=jnp.float32)], axis=0)

    idx_ref[0] = idxm + b * _N
    vt_ref[0] = vt
    ut_ref[0] = ut
    ht_ref[0] = ht
    st_ref[0] = st


def _stage_a(pts8, features, at, bmt, wsct):
    return pl.pallas_call(
        _event_body,
        grid=(_B,),
        in_specs=[
            pl.BlockSpec((1, 8, _N), lambda b: (b, 0, 0)),
            pl.BlockSpec((1, _DIN, _N), lambda b: (b, 0, 0)),
            pl.BlockSpec((_DIN, _DOUT), lambda b: (0, 0)),
            pl.BlockSpec((_DIN, _DOUT), lambda b: (0, 0)),
            pl.BlockSpec((_DIN, _DOUT), lambda b: (0, 0)),
        ],
        out_specs=[
            pl.BlockSpec((1, _N, _K), lambda b: (b, 0, 0)),
            pl.BlockSpec((1, _N, _DOUT), lambda b: (b, 0, 0)),
            pl.BlockSpec((1, _N, _DOUT), lambda b: (b, 0, 0)),
            pl.BlockSpec((1, _N, _DOUT), lambda b: (b, 0, 0)),
            pl.BlockSpec((1, 8, _DOUT), lambda b: (b, 0, 0)),
        ],
        out_shape=[
            jax.ShapeDtypeStruct((_B, _N, _K), jnp.int32),
            jax.ShapeDtypeStruct((_B, _N, _DOUT), jnp.float32),
            jax.ShapeDtypeStruct((_B, _N, _DOUT), jnp.float32),
            jax.ShapeDtypeStruct((_B, _N, _DOUT), jnp.float32),
            jax.ShapeDtypeStruct((_B, 8, _DOUT), jnp.float32),
        ],
        scratch_shapes=[
            pltpu.VMEM((_N, _N), jnp.float32),
            pltpu.VMEM((_N, _N), jnp.float32),
        ],
    )(pts8, features, at, bmt, wsct)


_NODES_PER_CHUNK = 8
_EDGES_PER_CHUNK = _NODES_PER_CHUNK * _K
_NUM_CHUNKS = (_B * _N) // _NODES_PER_CHUNK

@functools.cache
def _make_stage_b():
    vector_mesh = plsc.VectorSubcoreMesh(
        core_axis_name="c", subcore_axis_name="s")

    @functools.partial(
        pl.kernel,
        out_type=jax.ShapeDtypeStruct((_B * _N, _DOUT), jnp.float32),
        mesh=vector_mesh,
        scratch_types=[
            pltpu.VMEM((8, _DOUT), jnp.float32),
            pltpu.VMEM((_EDGES_PER_CHUNK, _DOUT), jnp.float32),
        ],
    )
    def stage_b(vt_hbm, ut_hbm, ht_hbm, fidx_hbm, consts_hbm, out_hbm,
                consts_v, rows_v):
        pltpu.sync_copy(consts_hbm, consts_v)

        def body(i_vmem, u_vmem, h_vmem, o_vmem):
            pltpu.sync_copy(vt_hbm.at[i_vmem.at[0]], rows_v)
            for n in range(_NODES_PER_CHUNK):
                for r in range(_DOUT // 16):
                    sl = pl.ds(r * 16, 16)
                    sc_g = consts_v[0, sl]
                    sh_g = consts_v[1, sl]
                    sc_h = consts_v[2, sl]
                    sh_h = consts_v[3, sl]
                    p = u_vmem[n, sl] * sc_g + sh_g
                    acc = jnp.zeros((16,), dtype=jnp.float32)
                    for k in range(_K):
                        v = rows_v[n * _K + k, sl]
                        acc = acc + jnp.maximum(v * sc_g + p, 0.0)
                    f = acc * (1.0 / _K)
                    hb = h_vmem[n, sl] * sc_h + sh_h
                    o_vmem[n, sl] = jnp.maximum(hb + f, 0.0)

        pltpu.emit_pipeline(
            body,
            grid=(_NUM_CHUNKS,),
            in_specs=[
                pl.BlockSpec((1, _EDGES_PER_CHUNK), lambda i: (0, i)),
                pl.BlockSpec((_NODES_PER_CHUNK, _DOUT), lambda i: (i, 0)),
                pl.BlockSpec((_NODES_PER_CHUNK, _DOUT), lambda i: (i, 0)),
            ],
            out_specs=[
                pl.BlockSpec((_NODES_PER_CHUNK, _DOUT), lambda i: (i, 0)),
            ],
            core_axis_name=("c", "s"),
            dimension_semantics=(pltpu.PARALLEL,),
        )(fidx_hbm, ut_hbm, ht_hbm, out_hbm)

    return stage_b


def kernel(points, features, W1, gamma1, beta1, Wsc, gamma_sc, beta_sc):
    a_mat = W1[:, :_DIN] - W1[:, _DIN:]
    bm_mat = W1[:, _DIN:]
    at, bmt, wsct = a_mat.T, bm_mat.T, Wsc.T
    pts8 = jnp.concatenate(
        [points, jnp.zeros((_B, 5, _N), dtype=points.dtype)], axis=1)

    fidx, vt, ut, ht, st = _stage_a(pts8, features, at, bmt, wsct)

    s = jnp.sum(st, axis=0)
    cnt_g = float(_B * _N * _K)
    mean_g = s[0] / cnt_g
    var_g = s[1] / cnt_g - mean_g * mean_g
    sc_g = gamma1 * lax.rsqrt(var_g + _EPS)
    sh_g = beta1 - mean_g * sc_g
    cnt_h = float(_B * _N)
    mean_h = s[2] / cnt_h
    var_h = s[3] / cnt_h - mean_h * mean_h
    sc_h = gamma_sc * lax.rsqrt(var_h + _EPS)
    sh_h = beta_sc - mean_h * sc_h
    consts = jnp.concatenate([
        sc_g[None, :], sh_g[None, :], sc_h[None, :], sh_h[None, :],
        jnp.zeros((4, _DOUT), dtype=jnp.float32)], axis=0)

    out_rows = _make_stage_b()(
        vt.reshape(_B * _N, _DOUT),
        ut.reshape(_B * _N, _DOUT),
        ht.reshape(_B * _N, _DOUT),
        fidx.reshape(1, _B * _N * _K),
        consts)
    return out_rows.reshape(_B, _N, _DOUT).transpose(0, 2, 1)


# speedup vs baseline: 7.5289x; 1.9770x over previous
import functools

import jax
import jax.numpy as jnp
from jax import lax
from jax.experimental import pallas as pl
from jax.experimental.pallas import tpu as pltpu
from jax.experimental.pallas import tpu_sc as plsc

_B, _N, _K, _DIN, _DOUT = 32, 1024, 16, 64, 128
_EPS = 1e-5
_SENT = -1e30
_SENT0 = -2e30


_EV = 4


def _event_body(pts_ref, x_ref, at_ref, bmt_ref, wsct_ref,
                idx_ref, vt_ref, ut_ref, ht_ref, st_ref, arr_ref):
    b = pl.program_id(0)

    ci = lax.broadcasted_iota(jnp.int32, (_N, _N), 1)
    ki = lax.broadcasted_iota(jnp.int32, (_N, _K), 1)
    li = lax.broadcasted_iota(jnp.int32, (_N, 128), 1)

    for e in range(_EV):
        pts = pts_ref[e]
        ptst = pts.T
        inner = jax.lax.dot_general(
            ptst, pts, (((1,), (0,)), ((), ())),
            preferred_element_type=jnp.float32)
        r_col = jnp.sum(ptst * ptst, axis=1, keepdims=True)
        r_row = jnp.sum(pts * pts, axis=0)[None, :]
        arr_ref[e] = 2.0 * inner - r_col - r_row
        idx_ref[e] = jnp.zeros((_N, _K), dtype=jnp.int32)

    def extract(prev_am, sent):
        cur = [None] * _EV
        cur_a = [None] * _EV
        for a in range(_N // 128):
            asl = pl.ds(a * 128, 128)
            for e in range(_EV):
                blk = arr_ref[e, :, asl]
                excl = (li + a * 128) == prev_am[e]
                blk = jnp.where(excl, sent, blk)
                arr_ref[e, :, asl] = blk
                if a == 0:
                    cur[e] = blk
                    cur_a[e] = jnp.zeros((_N, 128), dtype=jnp.int32)
                else:
                    m = blk > cur[e]
                    cur[e] = jnp.where(m, blk, cur[e])
                    cur_a[e] = jnp.where(m, a, cur_a[e])
        ams = []
        for e in range(_EV):
            mx = jnp.max(cur[e], axis=1, keepdims=True)
            am = jnp.min(jnp.where(cur[e] == mx, cur_a[e] * 128 + li, _N),
                         axis=1, keepdims=True)
            ams.append(am)
        return tuple(ams)

    none_am = jnp.full((_N, 1), -1, dtype=jnp.int32)
    am_self = extract((none_am,) * _EV, _SENT)
    am_first = extract(am_self, _SENT0)
    for e in range(_EV):
        idx_ref[e] = jnp.where(ki == 0, am_first[e], idx_ref[e])

    def step(t, prev_am):
        am = extract(prev_am, _SENT)
        for e in range(_EV):
            idx_ref[e] = jnp.where(ki == t, am[e], idx_ref[e])
        return am

    am_last = lax.fori_loop(1, _K, step, am_first)

    for e in range(_EV):
        m_adj = ((arr_ref[e] == _SENT) | (ci == am_last[e])).astype(
            jnp.float32)

        x = x_ref[e]
        xt = x.T
        ut = jnp.dot(xt, at_ref[...], preferred_element_type=jnp.float32)
        vt = jnp.dot(xt, bmt_ref[...], preferred_element_type=jnp.float32)
        ht = jnp.dot(xt, wsct_ref[...], preferred_element_type=jnp.float32)

        sq_nb = jnp.dot(m_adj, jnp.concatenate([vt, vt * vt], axis=1),
                        preferred_element_type=jnp.float32)
        s_nb = sq_nb[:, :_DOUT]
        q_nb = sq_nb[:, _DOUT:]

        sum_g = _K * jnp.sum(ut, axis=0) + jnp.sum(s_nb, axis=0)
        ssq_g = (_K * jnp.sum(ut * ut, axis=0)
                 + 2.0 * jnp.sum(ut * s_nb, axis=0)
                 + jnp.sum(q_nb, axis=0))
        sum_h = jnp.sum(ht, axis=0)
        ssq_h = jnp.sum(ht * ht, axis=0)
        st = jnp.concatenate([
            sum_g[None, :], ssq_g[None, :], sum_h[None, :], ssq_h[None, :],
            jnp.zeros((4, _DOUT), dtype=jnp.float32)], axis=0)

        idx_ref[e] = idx_ref[e] + (b * _EV + e) * _N
        vt_ref[e] = vt
        ut_ref[e] = ut
        ht_ref[e] = ht
        st_ref[e] = st


def _stage_a(pts8, features, at, bmt, wsct):
    return pl.pallas_call(
        _event_body,
        grid=(_B // _EV,),
        in_specs=[
            pl.BlockSpec((_EV, 8, _N), lambda b: (b, 0, 0)),
            pl.BlockSpec((_EV, _DIN, _N), lambda b: (b, 0, 0)),
            pl.BlockSpec((_DIN, _DOUT), lambda b: (0, 0)),
            pl.BlockSpec((_DIN, _DOUT), lambda b: (0, 0)),
            pl.BlockSpec((_DIN, _DOUT), lambda b: (0, 0)),
        ],
        out_specs=[
            pl.BlockSpec((_EV, _N, _K), lambda b: (b, 0, 0)),
            pl.BlockSpec((_EV, _N, _DOUT), lambda b: (b, 0, 0)),
            pl.BlockSpec((_EV, _N, _DOUT), lambda b: (b, 0, 0)),
            pl.BlockSpec((_EV, _N, _DOUT), lambda b: (b, 0, 0)),
            pl.BlockSpec((_EV, 8, _DOUT), lambda b: (b, 0, 0)),
        ],
        out_shape=[
            jax.ShapeDtypeStruct((_B, _N, _K), jnp.int32),
            jax.ShapeDtypeStruct((_B, _N, _DOUT), jnp.float32),
            jax.ShapeDtypeStruct((_B, _N, _DOUT), jnp.float32),
            jax.ShapeDtypeStruct((_B, _N, _DOUT), jnp.float32),
            jax.ShapeDtypeStruct((_B, 8, _DOUT), jnp.float32),
        ],
        scratch_shapes=[
            pltpu.VMEM((_EV, _N, _N), jnp.float32),
        ],
    )(pts8, features, at, bmt, wsct)


_NODES_PER_CHUNK = 8
_EDGES_PER_CHUNK = _NODES_PER_CHUNK * _K
_NUM_CHUNKS = (_B * _N) // _NODES_PER_CHUNK

_CHUNKS_PER_EVENT = _N // _NODES_PER_CHUNK


@functools.cache
def _make_stage_b():
    vector_mesh = plsc.VectorSubcoreMesh(
        core_axis_name="c", subcore_axis_name="s")

    @functools.partial(
        pl.kernel,
        out_type=jax.ShapeDtypeStruct((_B * _N, _DOUT), jnp.float32),
        mesh=vector_mesh,
        scratch_types=[
            pltpu.VMEM((_N * _K,), jnp.int32),
            pltpu.VMEM((8, _DOUT), jnp.float32),
            pltpu.VMEM((2, _EDGES_PER_CHUNK, _DOUT), jnp.float32),
            pltpu.VMEM((2, _NODES_PER_CHUNK, _DOUT), jnp.float32),
            pltpu.VMEM((2, _NODES_PER_CHUNK, _DOUT), jnp.float32),
            pltpu.VMEM((2, _NODES_PER_CHUNK, _DOUT), jnp.float32),
            pltpu.SemaphoreType.DMA((2,)),
            pltpu.SemaphoreType.DMA((2,)),
            pltpu.SemaphoreType.DMA((2,)),
            pltpu.SemaphoreType.DMA((2,)),
        ],
    )
    def stage_b(vt_hbm, ut_hbm, ht_hbm, fidx_hbm, consts_hbm, out_hbm,
                idx_all, consts_v, rows2, u2, h2, o2,
                sem_g, sem_u, sem_h, sem_o):
        w = lax.axis_index("s") * 2 + lax.axis_index("c")
        ebase = w * _N
        pltpu.sync_copy(consts_hbm, consts_v)
        pltpu.sync_copy(fidx_hbm.at[pl.ds(ebase * _K, _N * _K)], idx_all)

        def g_copy(c, p):
            return pltpu.make_async_copy(
                vt_hbm.at[idx_all.at[pl.ds(c * _EDGES_PER_CHUNK,
                                           _EDGES_PER_CHUNK)]],
                rows2.at[p], sem_g.at[p])

        def u_copy(c, p):
            return pltpu.make_async_copy(
                ut_hbm.at[pl.ds(ebase + c * _NODES_PER_CHUNK,
                                _NODES_PER_CHUNK)], u2.at[p], sem_u.at[p])

        def h_copy(c, p):
            return pltpu.make_async_copy(
                ht_hbm.at[pl.ds(ebase + c * _NODES_PER_CHUNK,
                                _NODES_PER_CHUNK)], h2.at[p], sem_h.at[p])

        def o_copy(c, p):
            return pltpu.make_async_copy(
                o2.at[p],
                out_hbm.at[pl.ds(ebase + c * _NODES_PER_CHUNK,
                                 _NODES_PER_CHUNK)], sem_o.at[p])

        def start_in(c, p):
            g_copy(c, p).start()
            u_copy(c, p).start()
            h_copy(c, p).start()

        def wait_in(c, p):
            g_copy(c, p).wait()
            u_copy(c, p).wait()
            h_copy(c, p).wait()

        def compute(p):
            for r in range(_DOUT // 16):
                sl = pl.ds(r * 16, 16)
                sc_g = consts_v[0, sl]
                sh_g = consts_v[1, sl]
                sc_h = consts_v[2, sl]
                sh_h = consts_v[3, sl]

                @pl.loop(0, _NODES_PER_CHUNK)
                def _(n):
                    pv = u2[p, n, sl] * sc_g + sh_g
                    acc = jnp.zeros((16,), dtype=jnp.float32)
                    for k in range(_K):
                        v = rows2[p, n * _K + k, sl]
                        acc = acc + jnp.maximum(v * sc_g + pv, 0.0)
                    hb = h2[p, n, sl] * sc_h + sh_h
                    o2[p, n, sl] = jnp.maximum(hb + acc * (1.0 / _K), 0.0)

        start_in(0, 0)

        @pl.loop(0, _CHUNKS_PER_EVENT // 2)
        def _(s):
            c0 = s * 2
            start_in(c0 + 1, 1)
            wait_in(c0, 0)

            @pl.when(s > 0)
            def _():
                o_copy(c0 - 2, 0).wait()

            compute(0)
            o_copy(c0, 0).start()

            @pl.when(s < _CHUNKS_PER_EVENT // 2 - 1)
            def _():
                start_in(c0 + 2, 0)

            wait_in(c0 + 1, 1)

            @pl.when(s > 0)
            def _():
                o_copy(c0 - 1, 1).wait()

            compute(1)
            o_copy(c0 + 1, 1).start()

        o_copy(_CHUNKS_PER_EVENT - 2, 0).wait()
        o_copy(_CHUNKS_PER_EVENT - 1, 1).wait()

    return stage_b


def kernel(points, features, W1, gamma1, beta1, Wsc, gamma_sc, beta_sc):
    a_mat = W1[:, :_DIN] - W1[:, _DIN:]
    bm_mat = W1[:, _DIN:]
    at, bmt, wsct = a_mat.T, bm_mat.T, Wsc.T
    pts8 = jnp.concatenate(
        [points, jnp.zeros((_B, 5, _N), dtype=points.dtype)], axis=1)

    fidx, vt, ut, ht, st = _stage_a(pts8, features, at, bmt, wsct)

    s = jnp.sum(st, axis=0)
    cnt_g = float(_B * _N * _K)
    mean_g = s[0] / cnt_g
    var_g = s[1] / cnt_g - mean_g * mean_g
    sc_g = gamma1 * lax.rsqrt(var_g + _EPS)
    sh_g = beta1 - mean_g * sc_g
    cnt_h = float(_B * _N)
    mean_h = s[2] / cnt_h
    var_h = s[3] / cnt_h - mean_h * mean_h
    sc_h = gamma_sc * lax.rsqrt(var_h + _EPS)
    sh_h = beta_sc - mean_h * sc_h
    consts = jnp.concatenate([
        sc_g[None, :], sh_g[None, :], sc_h[None, :], sh_h[None, :],
        jnp.zeros((4, _DOUT), dtype=jnp.float32)], axis=0)

    out_rows = _make_stage_b()(
        vt.reshape(_B * _N, _DOUT),
        ut.reshape(_B * _N, _DOUT),
        ht.reshape(_B * _N, _DOUT),
        fidx.reshape(_B * _N * _K),
        consts)
    return out_rows.reshape(_B, _N, _DOUT).transpose(0, 2, 1)


# speedup vs baseline: 7.5308x; 1.0003x over previous
import functools

import jax
import jax.numpy as jnp
from jax import lax
from jax.experimental import pallas as pl
from jax.experimental.pallas import tpu as pltpu
from jax.experimental.pallas import tpu_sc as plsc

_B, _N, _K, _DIN, _DOUT = 32, 1024, 16, 64, 128
_EPS = 1e-5
_SENT = -1e30
_SENT0 = -2e30


_EV = 4


def _event_body(pts_ref, x_ref, at_ref, bmt_ref, wsct_ref,
                idx_ref, vt_ref, ut_ref, ht_ref, st_ref, arr_ref):
    b = pl.program_id(0)

    ci = lax.broadcasted_iota(jnp.int32, (_N, _N), 1)
    ki = lax.broadcasted_iota(jnp.int32, (_N, _K), 1)
    li = lax.broadcasted_iota(jnp.int32, (_N, 128), 1)

    for e in range(_EV):
        pts = pts_ref[e]
        ptst = pts.T
        inner = jax.lax.dot_general(
            ptst, pts, (((1,), (0,)), ((), ())),
            preferred_element_type=jnp.float32)
        r_col = jnp.sum(ptst * ptst, axis=1, keepdims=True)
        r_row = jnp.sum(pts * pts, axis=0)[None, :]
        arr_ref[e] = 2.0 * inner - r_col - r_row
        idx_ref[e] = jnp.zeros((_N, _K), dtype=jnp.int32)

    def extract(prev_am, sent):
        cur = [None] * _EV
        cur_a = [None] * _EV
        for a in range(_N // 128):
            asl = pl.ds(a * 128, 128)
            for e in range(_EV):
                blk = arr_ref[e, :, asl]
                excl = (li + a * 128) == prev_am[e]
                blk = jnp.where(excl, sent, blk)
                arr_ref[e, :, asl] = blk
                if a == 0:
                    cur[e] = blk
                    cur_a[e] = jnp.zeros((_N, 128), dtype=jnp.int32)
                else:
                    m = blk > cur[e]
                    cur[e] = jnp.where(m, blk, cur[e])
                    cur_a[e] = jnp.where(m, a, cur_a[e])
        ams = []
        for e in range(_EV):
            mx = jnp.max(cur[e], axis=1, keepdims=True)
            am = jnp.min(jnp.where(cur[e] == mx, cur_a[e] * 128 + li, _N),
                         axis=1, keepdims=True)
            ams.append(am)
        return tuple(ams)

    none_am = jnp.full((_N, 1), -1, dtype=jnp.int32)
    am_self = extract((none_am,) * _EV, _SENT)
    am_first = extract(am_self, _SENT0)
    for e in range(_EV):
        idx_ref[e] = jnp.where(ki == 0, am_first[e], idx_ref[e])

    def step(t, prev_am):
        am = extract(prev_am, _SENT)
        for e in range(_EV):
            idx_ref[e] = jnp.where(ki == t, am[e], idx_ref[e])
        return am

    am_last = lax.fori_loop(1, _K, step, am_first)

    for e in range(_EV):
        m_adj = ((arr_ref[e] == _SENT) | (ci == am_last[e])).astype(
            jnp.float32)

        x = x_ref[e]
        xt = x.T
        ut = jnp.dot(xt, at_ref[...], preferred_element_type=jnp.float32)
        vt = jnp.dot(xt, bmt_ref[...], preferred_element_type=jnp.float32)
        ht = jnp.dot(xt, wsct_ref[...], preferred_element_type=jnp.float32)

        sq_nb = jnp.dot(m_adj, jnp.concatenate([vt, vt * vt], axis=1),
                        preferred_element_type=jnp.float32)
        s_nb = sq_nb[:, :_DOUT]
        q_nb = sq_nb[:, _DOUT:]

        sum_g = _K * jnp.sum(ut, axis=0) + jnp.sum(s_nb, axis=0)
        ssq_g = (_K * jnp.sum(ut * ut, axis=0)
                 + 2.0 * jnp.sum(ut * s_nb, axis=0)
                 + jnp.sum(q_nb, axis=0))
        sum_h = jnp.sum(ht, axis=0)
        ssq_h = jnp.sum(ht * ht, axis=0)
        st = jnp.concatenate([
            sum_g[None, :], ssq_g[None, :], sum_h[None, :], ssq_h[None, :],
            jnp.zeros((4, _DOUT), dtype=jnp.float32)], axis=0)

        idx_ref[e] = idx_ref[e] + (b * _EV + e) * _N
        vt_ref[e] = vt
        ut_ref[e] = ut
        ht_ref[e] = ht
        st_ref[e] = st


def _stage_a(pts8, features, at, bmt, wsct):
    return pl.pallas_call(
        _event_body,
        grid=(_B // _EV,),
        in_specs=[
            pl.BlockSpec((_EV, 8, _N), lambda b: (b, 0, 0)),
            pl.BlockSpec((_EV, _DIN, _N), lambda b: (b, 0, 0)),
            pl.BlockSpec((_DIN, _DOUT), lambda b: (0, 0)),
            pl.BlockSpec((_DIN, _DOUT), lambda b: (0, 0)),
            pl.BlockSpec((_DIN, _DOUT), lambda b: (0, 0)),
        ],
        out_specs=[
            pl.BlockSpec((_EV, _N, _K), lambda b: (b, 0, 0)),
            pl.BlockSpec((_EV, _N, _DOUT), lambda b: (b, 0, 0)),
            pl.BlockSpec((_EV, _N, _DOUT), lambda b: (b, 0, 0)),
            pl.BlockSpec((_EV, _N, _DOUT), lambda b: (b, 0, 0)),
            pl.BlockSpec((_EV, 8, _DOUT), lambda b: (b, 0, 0)),
        ],
        out_shape=[
            jax.ShapeDtypeStruct((_B, _N, _K), jnp.int32),
            jax.ShapeDtypeStruct((_B, _N, _DOUT), jnp.float32),
            jax.ShapeDtypeStruct((_B, _N, _DOUT), jnp.float32),
            jax.ShapeDtypeStruct((_B, _N, _DOUT), jnp.float32),
            jax.ShapeDtypeStruct((_B, 8, _DOUT), jnp.float32),
        ],
        scratch_shapes=[
            pltpu.VMEM((_EV, _N, _N), jnp.float32),
        ],
    )(pts8, features, at, bmt, wsct)


_NODES_PER_CHUNK = 8
_EDGES_PER_CHUNK = _NODES_PER_CHUNK * _K
_CHUNKS_PER_EVENT = _N // _NODES_PER_CHUNK


@functools.cache
def _make_stage_b():
    vector_mesh = plsc.VectorSubcoreMesh(
        core_axis_name="c", subcore_axis_name="s")

    @functools.partial(
        pl.kernel,
        out_type=jax.ShapeDtypeStruct((_B * _N, _DOUT), jnp.float32),
        mesh=vector_mesh,
        scratch_types=[
            pltpu.VMEM((_N * _K,), jnp.int32),
            pltpu.VMEM((8, _DOUT), jnp.float32),
            pltpu.VMEM((2, _EDGES_PER_CHUNK, _DOUT), jnp.float32),
            pltpu.VMEM((2, _NODES_PER_CHUNK, _DOUT), jnp.float32),
            pltpu.VMEM((2, _NODES_PER_CHUNK, _DOUT), jnp.float32),
            pltpu.VMEM((2, _NODES_PER_CHUNK, _DOUT), jnp.float32),
            pltpu.SemaphoreType.DMA((2,)),
            pltpu.SemaphoreType.DMA((2,)),
            pltpu.SemaphoreType.DMA((2,)),
            pltpu.SemaphoreType.DMA((2,)),
        ],
    )
    def stage_b(vt_hbm, ut_hbm, ht_hbm, fidx_hbm, consts_hbm, out_hbm,
                idx_all, consts_v, rows2, u2, h2, o2,
                sem_g, sem_u, sem_h, sem_o):
        w = lax.axis_index("s") * 2 + lax.axis_index("c")
        ebase = w * _N
        pltpu.sync_copy(consts_hbm, consts_v)
        pltpu.sync_copy(fidx_hbm.at[pl.ds(ebase * _K, _N * _K)], idx_all)

        def g_copy(c, p):
            return pltpu.make_async_copy(
                vt_hbm.at[idx_all.at[pl.ds(c * _EDGES_PER_CHUNK,
                                           _EDGES_PER_CHUNK)]],
                rows2.at[p], sem_g.at[p])

        def u_copy(c, p):
            return pltpu.make_async_copy(
                ut_hbm.at[pl.ds(ebase + c * _NODES_PER_CHUNK,
                                _NODES_PER_CHUNK)], u2.at[p], sem_u.at[p])

        def h_copy(c, p):
            return pltpu.make_async_copy(
                ht_hbm.at[pl.ds(ebase + c * _NODES_PER_CHUNK,
                                _NODES_PER_CHUNK)], h2.at[p], sem_h.at[p])

        def o_copy(c, p):
            return pltpu.make_async_copy(
                o2.at[p],
                out_hbm.at[pl.ds(ebase + c * _NODES_PER_CHUNK,
                                 _NODES_PER_CHUNK)], sem_o.at[p])

        def start_in(c, p):
            g_copy(c, p).start()
            u_copy(c, p).start()
            h_copy(c, p).start()

        def wait_in(c, p):
            g_copy(c, p).wait()
            u_copy(c, p).wait()
            h_copy(c, p).wait()

        def compute(p):
            for r in range(_DOUT // 16):
                sl = pl.ds(r * 16, 16)
                sc_g = consts_v[0, sl]
                sh_g = consts_v[1, sl]
                sc_h = consts_v[2, sl]
                sh_h = consts_v[3, sl]

                @pl.loop(0, _NODES_PER_CHUNK)
                def _(n):
                    pv = u2[p, n, sl] * sc_g + sh_g
                    acc = jnp.zeros((16,), dtype=jnp.float32)
                    for k in range(_K):
                        v = rows2[p, n * _K + k, sl]
                        acc = acc + jnp.maximum(v * sc_g + pv, 0.0)
                    hb = h2[p, n, sl] * sc_h + sh_h
                    o2[p, n, sl] = jnp.maximum(hb + acc * (1.0 / _K), 0.0)

        start_in(0, 0)

        @pl.loop(0, _CHUNKS_PER_EVENT // 2)
        def _(s):
            c0 = s * 2
            start_in(c0 + 1, 1)
            wait_in(c0, 0)

            @pl.when(s > 0)
            def _():
                o_copy(c0 - 2, 0).wait()

            compute(0)
            o_copy(c0, 0).start()

            @pl.when(s < _CHUNKS_PER_EVENT // 2 - 1)
            def _():
                start_in(c0 + 2, 0)

            wait_in(c0 + 1, 1)

            @pl.when(s > 0)
            def _():
                o_copy(c0 - 1, 1).wait()

            compute(1)
            o_copy(c0 + 1, 1).start()

        o_copy(_CHUNKS_PER_EVENT - 2, 0).wait()
        o_copy(_CHUNKS_PER_EVENT - 1, 1).wait()

    return stage_b


def kernel(points, features, W1, gamma1, beta1, Wsc, gamma_sc, beta_sc):
    a_mat = W1[:, :_DIN] - W1[:, _DIN:]
    bm_mat = W1[:, _DIN:]
    at, bmt, wsct = a_mat.T, bm_mat.T, Wsc.T
    pts8 = jnp.concatenate(
        [points, jnp.zeros((_B, 5, _N), dtype=points.dtype)], axis=1)

    fidx, vt, ut, ht, st = _stage_a(pts8, features, at, bmt, wsct)

    s = jnp.sum(st, axis=0)
    cnt_g = float(_B * _N * _K)
    mean_g = s[0] / cnt_g
    var_g = s[1] / cnt_g - mean_g * mean_g
    sc_g = gamma1 * lax.rsqrt(var_g + _EPS)
    sh_g = beta1 - mean_g * sc_g
    cnt_h = float(_B * _N)
    mean_h = s[2] / cnt_h
    var_h = s[3] / cnt_h - mean_h * mean_h
    sc_h = gamma_sc * lax.rsqrt(var_h + _EPS)
    sh_h = beta_sc - mean_h * sc_h
    consts = jnp.concatenate([
        sc_g[None, :], sh_g[None, :], sc_h[None, :], sh_h[None, :],
        jnp.zeros((4, _DOUT), dtype=jnp.float32)], axis=0)

    out_rows = _make_stage_b()(
        vt.reshape(_B * _N, _DOUT),
        ut.reshape(_B * _N, _DOUT),
        ht.reshape(_B * _N, _DOUT),
        fidx.reshape(_B * _N * _K),
        consts)
    return out_rows.reshape(_B, _N, _DOUT).transpose(0, 2, 1)
